# baseline (speedup 1.0000x reference)
import functools

import jax
import jax.numpy as jnp
from jax import lax
from jax.experimental import pallas as pl
from jax.experimental.pallas import tpu as pltpu
from jax.experimental.pallas import tpu_sc as plsc

N_NODES = 10000
N_EDGES = 320000
D = 128
LANES = 16
N_FB = D // LANES
N_RANGES = 4
NODES_PER_RANGE = N_NODES // N_RANGES
CHUNK = 3200
N_CHUNKS = N_EDGES // CHUNK

_INF = float("inf")


N_CORES = 2
N_SUBCORES = 16


def _sc_body(src_hbm, dst_hbm, out_hbm, idx_v, row_v, acc_v):
    wid = lax.axis_index("s") * N_CORES + lax.axis_index("c")
    fb = wid % N_FB
    nr = wid // N_FB
    base = nr * NODES_PER_RANGE

    def init_body(i, _):
        acc_v[i] = jnp.full((LANES,), _INF, jnp.float32)
        return 0
    lax.fori_loop(0, NODES_PER_RANGE + 1, init_body, 0)

    def chunk_body(c, _):
        e0 = c * CHUNK
        pltpu.sync_copy(dst_hbm.at[pl.ds(e0, CHUNK)], idx_v)
        pltpu.sync_copy(src_hbm.at[pl.ds(e0, CHUNK), fb], row_v)

        def group_body(g, _):
            d16 = idx_v[pl.ds(g * LANES, LANES)]
            loc16 = d16 - base
            ok16 = (loc16 >= 0) & (loc16 < NODES_PER_RANGE)
            li16 = jnp.where(ok16, loc16, NODES_PER_RANGE)
            for l in range(LANES):
                li = li16[l]
                e = g * LANES + l
                acc_v[li] = jnp.minimum(acc_v[li], row_v[e])
            return 0

        lax.fori_loop(0, CHUNK // LANES, group_body, 0)
        return 0

    lax.fori_loop(0, N_CHUNKS, chunk_body, 0)

    def fin_body(i, _):
        v = acc_v[i]
        acc_v[i] = jnp.where(v == _INF, jnp.float32(0.0), v)
        return 0
    lax.fori_loop(0, NODES_PER_RANGE, fin_body, 0)

    pltpu.sync_copy(acc_v.at[pl.ds(0, NODES_PER_RANGE)],
                    out_hbm.at[pl.ds(base, NODES_PER_RANGE), fb])


@jax.jit
def _segment_min_sc(src3, dst):
    mesh = plsc.VectorSubcoreMesh(
        core_axis_name="c", subcore_axis_name="s",
        num_cores=N_CORES, num_subcores=N_SUBCORES)
    return pl.kernel(
        _sc_body,
        out_type=jax.ShapeDtypeStruct((N_NODES, N_FB, LANES), jnp.float32),
        mesh=mesh,
        scratch_types=[
            pltpu.VMEM((CHUNK,), jnp.int32),
            pltpu.VMEM((CHUNK, LANES), jnp.float32),
            pltpu.VMEM((NODES_PER_RANGE + 1, LANES), jnp.float32),
        ],
        compiler_params=pltpu.CompilerParams(use_tc_tiling_on_sc=False),
    )(src3, dst)


def kernel(source_node_representation_with_coefficient, edge_index, feature_dim):
    src3 = source_node_representation_with_coefficient.reshape(N_EDGES, N_FB, LANES)
    dst = edge_index[1]
    out3 = _segment_min_sc(src3, dst)
    return out3.reshape(N_NODES, D)
